# Initial kernel scaffold; baseline (speedup 1.0000x reference)
import functools

import jax
import jax.numpy as jnp
from jax import lax
from jax.experimental import pallas as pl
from jax.experimental.pallas import tpu as pltpu
from jax.experimental.pallas import tpu_sc as plsc


def _pooling_kernel(B, L, V, E):
  mesh = plsc.VectorSubcoreMesh(core_axis_name="c", subcore_axis_name="s")
  NC, NS = mesh.num_cores, mesh.num_subcores
  NW = NC * NS
  assert B % NW == 0 and E == 32 and L % 8 == 0
  b_per_w = B // NW
  C0 = 104 if L > 128 else L
  C1 = L - C0
  assert C0 % 8 == 0 and C1 % 8 == 0 and C0 <= 128 and C1 <= 128

  @functools.partial(
      pl.kernel,
      out_type=jax.ShapeDtypeStruct((B, E), jnp.float32),
      mesh=mesh,
      scratch_types=[
          pltpu.VMEM((b_per_w * L,), jnp.int32),
          pltpu.VMEM((L, E), jnp.float32),
          pltpu.VMEM((L, E), jnp.float32),
          pltpu.VMEM((b_per_w, E), jnp.float32),
          pltpu.SemaphoreType.DMA,
          pltpu.SemaphoreType.DMA,
      ],
  )
  def kern(table_hbm, ids_hbm, pooled_hbm, ids_v, buf0, buf1, pooled_v,
           sem0, sem1):
    wid = lax.axis_index("s") * NC + lax.axis_index("c")
    base = wid * b_per_w

    pltpu.sync_copy(ids_hbm.at[pl.ds(base * L, b_per_w * L)], ids_v)

    def fire(i, buf, sem):
      off = i * L
      pltpu.async_copy(table_hbm.at[ids_v.at[pl.ds(off, C0)]],
                       buf.at[pl.ds(0, C0)], sem)
      if C1:
        pltpu.async_copy(table_hbm.at[ids_v.at[pl.ds(off + C0, C1)]],
                         buf.at[pl.ds(C0, C1)], sem)

    def wait(buf, sem):
      pltpu.make_async_copy(table_hbm.at[pl.ds(0, L)], buf, sem).wait()

    def acc_store(i, buf):
      zero = jnp.zeros((E // 2,), jnp.float32)
      h = E // 2

      def body(r, carry):
        a0, a1, b0, b1, c0, c1, d0, d1 = carry
        rr = r * 4
        a0 = a0 + buf[rr, pl.ds(0, h)]
        a1 = a1 + buf[rr, pl.ds(h, h)]
        b0 = b0 + buf[rr + 1, pl.ds(0, h)]
        b1 = b1 + buf[rr + 1, pl.ds(h, h)]
        c0 = c0 + buf[rr + 2, pl.ds(0, h)]
        c1 = c1 + buf[rr + 2, pl.ds(h, h)]
        d0 = d0 + buf[rr + 3, pl.ds(0, h)]
        d1 = d1 + buf[rr + 3, pl.ds(h, h)]
        return (a0, a1, b0, b1, c0, c1, d0, d1)

      a0, a1, b0, b1, c0, c1, d0, d1 = lax.fori_loop(
          0, L // 4, body, (zero,) * 8)
      pooled_v[i, pl.ds(0, h)] = (a0 + b0) + (c0 + d0)
      pooled_v[i, pl.ds(h, h)] = (a1 + b1) + (c1 + d1)

    fire(0, buf0, sem0)
    fire(1, buf1, sem1)

    def outer(j, carry):
      i = j * 2
      wait(buf0, sem0)
      acc_store(i, buf0)
      fire(i + 2, buf0, sem0)
      wait(buf1, sem1)
      acc_store(i + 1, buf1)
      fire(i + 3, buf1, sem1)
      return carry

    lax.fori_loop(0, b_per_w // 2 - 1, outer, 0)
    wait(buf0, sem0)
    acc_store(b_per_w - 2, buf0)
    wait(buf1, sem1)
    acc_store(b_per_w - 1, buf1)

    pltpu.sync_copy(pooled_v, pooled_hbm.at[pl.ds(base, b_per_w)])

  return kern


def _proj_body(p_ref, wt_ref, b_ref, o_ref):
  o_ref[...] = (
      jnp.dot(p_ref[...], wt_ref[...], preferred_element_type=jnp.float32)
      + b_ref[...])


def kernel(input_ids, table, W, b):
  B, L = input_ids.shape
  V, E = table.shape
  C = W.shape[0]
  ids_flat = input_ids.reshape(B * L).astype(jnp.int32)

  pooled = _pooling_kernel(B, L, V, E)(table, ids_flat)

  wt = W.T.astype(jnp.float32)
  bias = (jnp.float32(L) * b).reshape(1, C)
  out = pl.pallas_call(
      _proj_body,
      out_shape=jax.ShapeDtypeStruct((B, C), jnp.float32),
  )(pooled, wt, bias)
  return out


# speedup vs baseline: 34.7919x; 34.7919x over previous
import functools

import jax
import jax.numpy as jnp
from jax import lax
from jax.experimental import pallas as pl
from jax.experimental.pallas import tpu as pltpu
from jax.experimental.pallas import tpu_sc as plsc


def _pooling_kernel(B, L, V, E):
  mesh = plsc.VectorSubcoreMesh(core_axis_name="c", subcore_axis_name="s")
  NC, NS = mesh.num_cores, mesh.num_subcores
  NW = NC * NS
  assert B % NW == 0 and E == 32 and L % 8 == 0
  b_per_w = B // NW
  C0 = 104 if L > 128 else L
  C1 = L - C0
  assert C0 % 8 == 0 and C1 % 8 == 0 and C0 <= 128 and C1 <= 128

  @functools.partial(
      pl.kernel,
      out_type=jax.ShapeDtypeStruct((B, E), jnp.float32),
      mesh=mesh,
      scratch_types=[
          pltpu.VMEM((b_per_w * L,), jnp.int32),
          pltpu.VMEM((L, E), jnp.float32),
          pltpu.VMEM((L, E), jnp.float32),
          pltpu.VMEM((b_per_w, E), jnp.float32),
          pltpu.SemaphoreType.DMA,
          pltpu.SemaphoreType.DMA,
      ],
      compiler_params=pltpu.CompilerParams(use_tc_tiling_on_sc=False),
  )
  def kern(table_hbm, ids_hbm, pooled_hbm, ids_v, buf0, buf1, pooled_v,
           sem0, sem1):
    wid = lax.axis_index("s") * NC + lax.axis_index("c")
    base = wid * b_per_w

    pltpu.sync_copy(ids_hbm.at[pl.ds(base * L, b_per_w * L)], ids_v)

    def fire(i, buf, sem):
      off = i * L
      pltpu.async_copy(table_hbm.at[ids_v.at[pl.ds(off, C0)]],
                       buf.at[pl.ds(0, C0)], sem)
      if C1:
        pltpu.async_copy(table_hbm.at[ids_v.at[pl.ds(off + C0, C1)]],
                         buf.at[pl.ds(C0, C1)], sem)

    def wait(buf, sem):
      pltpu.make_async_copy(table_hbm.at[pl.ds(0, L)], buf, sem).wait()

    def acc_store(i, buf):
      zero = jnp.zeros((E // 2,), jnp.float32)
      h = E // 2

      def body(r, carry):
        a0, a1, b0, b1, c0, c1, d0, d1 = carry
        rr = r * 4
        a0 = a0 + buf[rr, pl.ds(0, h)]
        a1 = a1 + buf[rr, pl.ds(h, h)]
        b0 = b0 + buf[rr + 1, pl.ds(0, h)]
        b1 = b1 + buf[rr + 1, pl.ds(h, h)]
        c0 = c0 + buf[rr + 2, pl.ds(0, h)]
        c1 = c1 + buf[rr + 2, pl.ds(h, h)]
        d0 = d0 + buf[rr + 3, pl.ds(0, h)]
        d1 = d1 + buf[rr + 3, pl.ds(h, h)]
        return (a0, a1, b0, b1, c0, c1, d0, d1)

      a0, a1, b0, b1, c0, c1, d0, d1 = lax.fori_loop(
          0, L // 4, body, (zero,) * 8)
      pooled_v[i, pl.ds(0, h)] = (a0 + b0) + (c0 + d0)
      pooled_v[i, pl.ds(h, h)] = (a1 + b1) + (c1 + d1)

    fire(0, buf0, sem0)
    fire(1, buf1, sem1)

    def outer(j, carry):
      i = j * 2
      wait(buf0, sem0)
      acc_store(i, buf0)
      fire(i + 2, buf0, sem0)
      wait(buf1, sem1)
      acc_store(i + 1, buf1)
      fire(i + 3, buf1, sem1)
      return carry

    lax.fori_loop(0, b_per_w // 2 - 1, outer, 0)
    wait(buf0, sem0)
    acc_store(b_per_w - 2, buf0)
    wait(buf1, sem1)
    acc_store(b_per_w - 1, buf1)

    pltpu.sync_copy(pooled_v, pooled_hbm.at[pl.ds(base, b_per_w)])

  return kern


def _proj_body(p_ref, wt_ref, b_ref, o_ref):
  o_ref[...] = (
      jnp.dot(p_ref[...], wt_ref[...], preferred_element_type=jnp.float32)
      + b_ref[...])


def kernel(input_ids, table, W, b):
  B, L = input_ids.shape
  V, E = table.shape
  C = W.shape[0]
  ids_flat = input_ids.reshape(B * L).astype(jnp.int32)

  pooled = _pooling_kernel(B, L, V, E)(table, ids_flat)

  wt = W.T.astype(jnp.float32)
  bias = (jnp.float32(L) * b).reshape(1, C)
  out = pl.pallas_call(
      _proj_body,
      out_shape=jax.ShapeDtypeStruct((B, C), jnp.float32),
  )(pooled, wt, bias)
  return out


# speedup vs baseline: 36.5490x; 1.0505x over previous
import functools

import jax
import jax.numpy as jnp
from jax import lax
from jax.experimental import pallas as pl
from jax.experimental.pallas import tpu as pltpu
from jax.experimental.pallas import tpu_sc as plsc


def _pooling_kernel(B, L, V, E):
  mesh = plsc.VectorSubcoreMesh(core_axis_name="c", subcore_axis_name="s")
  NC, NS = mesh.num_cores, mesh.num_subcores
  NW = NC * NS
  assert B % NW == 0 and E == 32 and L % 8 == 0
  b_per_w = B // NW
  GROUP = 4
  CH = GROUP * L
  ngroups = b_per_w // GROUP
  assert b_per_w % GROUP == 0 and ngroups % 2 == 0
  chunks = []
  off = 0
  while off < CH:
    sz = min(128, CH - off)
    chunks.append((off, sz))
    off += sz
  assert all(o % 8 == 0 and s % 8 == 0 for o, s in chunks)

  @functools.partial(
      pl.kernel,
      out_type=jax.ShapeDtypeStruct((B, E), jnp.float32),
      mesh=mesh,
      scratch_types=[
          pltpu.VMEM((b_per_w * L,), jnp.int32),
          pltpu.VMEM((CH, E), jnp.float32),
          pltpu.VMEM((CH, E), jnp.float32),
          pltpu.VMEM((b_per_w, E), jnp.float32),
          pltpu.SemaphoreType.DMA,
          pltpu.SemaphoreType.DMA,
      ],
      compiler_params=pltpu.CompilerParams(use_tc_tiling_on_sc=False),
  )
  def kern(table_hbm, ids_hbm, pooled_hbm, ids_v, buf0, buf1, pooled_v,
           sem0, sem1):
    wid = lax.axis_index("s") * NC + lax.axis_index("c")
    base = wid * b_per_w

    pltpu.sync_copy(ids_hbm.at[pl.ds(base * L, b_per_w * L)], ids_v)

    def fire(g, buf, sem):
      gb = g * CH
      for o, s in chunks:
        pltpu.async_copy(table_hbm.at[ids_v.at[pl.ds(gb + o, s)]],
                         buf.at[pl.ds(o, s)], sem)

    def wait(buf, sem):
      pltpu.make_async_copy(table_hbm.at[pl.ds(0, CH)], buf, sem).wait()

    h = E // 2

    def acc_store(i, buf, row_off):
      zero = jnp.zeros((h,), jnp.float32)

      def body(r, carry):
        a0, a1, b0, b1, c0, c1, d0, d1 = carry
        rr = row_off + r * 4
        a0 = a0 + buf[rr, pl.ds(0, h)]
        a1 = a1 + buf[rr, pl.ds(h, h)]
        b0 = b0 + buf[rr + 1, pl.ds(0, h)]
        b1 = b1 + buf[rr + 1, pl.ds(h, h)]
        c0 = c0 + buf[rr + 2, pl.ds(0, h)]
        c1 = c1 + buf[rr + 2, pl.ds(h, h)]
        d0 = d0 + buf[rr + 3, pl.ds(0, h)]
        d1 = d1 + buf[rr + 3, pl.ds(h, h)]
        return (a0, a1, b0, b1, c0, c1, d0, d1)

      a0, a1, b0, b1, c0, c1, d0, d1 = lax.fori_loop(
          0, L // 4, body, (zero,) * 8)
      pooled_v[i, pl.ds(0, h)] = (a0 + b0) + (c0 + d0)
      pooled_v[i, pl.ds(h, h)] = (a1 + b1) + (c1 + d1)

    def acc_group(g, buf):
      for k in range(GROUP):
        acc_store(g * GROUP + k, buf, k * L)

    fire(0, buf0, sem0)
    fire(1, buf1, sem1)

    def outer(j, carry):
      g = j * 2
      wait(buf0, sem0)
      acc_group(g, buf0)
      fire(g + 2, buf0, sem0)
      wait(buf1, sem1)
      acc_group(g + 1, buf1)
      fire(g + 3, buf1, sem1)
      return carry

    lax.fori_loop(0, ngroups // 2 - 1, outer, 0)
    wait(buf0, sem0)
    acc_group(ngroups - 2, buf0)
    wait(buf1, sem1)
    acc_group(ngroups - 1, buf1)

    pltpu.sync_copy(pooled_v, pooled_hbm.at[pl.ds(base, b_per_w)])

  return kern


def _proj_body(p_ref, wt_ref, b_ref, o_ref):
  o_ref[...] = (
      jnp.dot(p_ref[...], wt_ref[...], preferred_element_type=jnp.float32)
      + b_ref[...])


def kernel(input_ids, table, W, b):
  B, L = input_ids.shape
  V, E = table.shape
  C = W.shape[0]
  ids_flat = input_ids.reshape(B * L).astype(jnp.int32)

  pooled = _pooling_kernel(B, L, V, E)(table, ids_flat)

  wt = W.T.astype(jnp.float32)
  bias = (jnp.float32(L) * b).reshape(1, C)
  out = pl.pallas_call(
      _proj_body,
      out_shape=jax.ShapeDtypeStruct((B, C), jnp.float32),
  )(pooled, wt, bias)
  return out
